# Initial kernel scaffold; baseline (speedup 1.0000x reference)
import jax
import jax.numpy as jnp
from jax.experimental import pallas as pl


def kernel(input_ids, logits):
    raise NotImplementedError("write your pallas kernel here")


# speedup vs baseline: 27.4805x; 27.4805x over previous
import jax
import jax.numpy as jnp
from jax.experimental import pallas as pl

_VOCAB = 100000
_SEED = 42
_PRIOR_TOKENS = 5
_K = 4
_TOP_P = 0.9


def _make_xi(input_ids):
    B = input_ids.shape[0]
    prior_ids = jnp.sum(input_ids[:, -_PRIOR_TOKENS:], axis=1).astype(jnp.uint32)

    def one(b, pid):
        hk = jax.random.fold_in(jax.random.key(_SEED + 1), b)
        hash_idx = jax.random.randint(hk, (), 0, _K)
        k = jax.random.key(_SEED)
        k = jax.random.fold_in(k, hash_idx)
        k = jax.random.fold_in(k, pid)
        xi = jax.random.uniform(k, (_VOCAB,), dtype=jnp.float32)
        return jnp.maximum(xi, 1e-12)

    return jax.vmap(one)(jnp.arange(B), prior_ids)


def _body(logits_ref, xi_ref, out_ref):
    l = logits_ref[...]
    m = jnp.max(l, axis=1, keepdims=True)
    e = jnp.exp(l - m)
    s = jnp.sum(e, axis=1, keepdims=True)
    theta = _TOP_P * s

    keys = jax.lax.bitcast_convert_type(e, jnp.int32)

    B = l.shape[0]
    lo0 = jnp.zeros((B, 1), jnp.int32)
    hi0 = jnp.max(keys, axis=1, keepdims=True) + 1

    def step(_, carry):
        lo, hi = carry
        mid = lo + jax.lax.shift_right_logical(hi - lo, 1)
        mass = jnp.sum(jnp.where(keys >= mid, e, 0.0), axis=1, keepdims=True)
        ok = mass >= theta
        return jnp.where(ok, mid, lo), jnp.where(ok, hi, mid)

    lo, hi = jax.lax.fori_loop(0, 31, step, (lo0, hi0))
    mask = keys >= lo

    nlx = -jnp.log(xi_ref[...])
    score = jnp.where(mask, nlx / e, jnp.inf)
    best = jnp.min(score, axis=1, keepdims=True)
    iota = jax.lax.broadcasted_iota(jnp.int32, l.shape, 1)
    win = jnp.min(jnp.where(score == best, iota, _VOCAB), axis=1, keepdims=True)

    out_ref[...] = jnp.where(iota == win, 100000.0, -100000.0)


def kernel(input_ids, logits):
    xi = _make_xi(input_ids)
    return pl.pallas_call(
        _body,
        out_shape=jax.ShapeDtypeStruct(logits.shape, jnp.float32),
    )(logits, xi)
